# baseline (speedup 1.0000x reference)
import functools

import numpy as np
import jax
import jax.numpy as jnp
from jax import lax
from jax.experimental import pallas as pl
from jax.experimental.pallas import tpu as pltpu
from jax.experimental.pallas import tpu_sc as plsc

_NC = 2
_NS = 16
_G = 128
_NB = 8


@functools.lru_cache(maxsize=None)
def _make_prop(n_pad, h, n_rows):
    assert n_rows % _NB == 0
    chunk = n_pad // _NS
    mesh = plsc.VectorSubcoreMesh(core_axis_name="c", subcore_axis_name="s")

    @functools.partial(
        pl.kernel,
        mesh=mesh,
        out_type=[
            jax.ShapeDtypeStruct((n_pad, h), jnp.float32),
            jax.ShapeDtypeStruct((n_pad, h), jnp.float32),
        ],
        scratch_types=[
            pltpu.VMEM((n_rows, _G), jnp.int32),
            pltpu.VMEM((n_rows, _G), jnp.int32),
            pltpu.VMEM((_NB, _G, h), jnp.float32),
            pltpu.VMEM_SHARED((n_pad, h), jnp.float32),
            pltpu.VMEM_SHARED((n_pad, h), jnp.float32),
            pltpu.SemaphoreType.DMA((_NB,)),
            pltpu.SemaphoreType.DMA((_NB,)),
        ],
        compiler_params=pltpu.CompilerParams(use_tc_tiling_on_sc=False),
    )
    def prop(edges_hbm, g_hbm, out0_hbm, out1_hbm,
             src_a, dst_a, rows, acc, gtab, semg, sems):
        cid = lax.axis_index("c")
        sid = lax.axis_index("s")
        wid = cid * _NS + sid
        r0 = sid * chunk
        row_base = wid * n_rows
        isem = semg.at[0]
        src_sl = edges_hbm.at[0, pl.ds(row_base, n_rows)]
        dst_sl = edges_hbm.at[1, pl.ds(row_base, n_rows)]
        pltpu.async_copy(src_sl, src_a, isem)
        pltpu.async_copy(dst_sl, dst_a, isem)
        pltpu.sync_copy(g_hbm.at[pl.ds(r0, chunk)], acc.at[pl.ds(r0, chunk)])
        pltpu.sync_copy(g_hbm.at[pl.ds(r0, chunk)], gtab.at[pl.ds(r0, chunk)])
        pltpu.make_async_copy(src_sl, src_a, isem).wait()
        pltpu.make_async_copy(dst_sl, dst_a, isem).wait()
        plsc.subcore_barrier()

        def gstart(i, b):
            pltpu.async_copy(gtab.at[src_a.at[i]], rows.at[b], semg.at[b])

        def gwait(b):
            pltpu.make_async_copy(gtab.at[src_a.at[0]], rows.at[b],
                                  semg.at[b]).wait()

        def sstart(i, b):
            pltpu.async_copy(rows.at[b], acc.at[dst_a.at[i]], sems.at[b],
                             add=True)

        def swait(b):
            pltpu.make_async_copy(rows.at[b], acc.at[dst_a.at[0]],
                                  sems.at[b]).wait()

        for b in range(_NB):
            gstart(b, b)

        def step(k, carry):
            i = _NB * k
            for b in range(_NB):
                gwait(b)
                sstart(i + b, b)
            for b in range(_NB):
                swait(b)
                gstart(i + _NB + b, b)
            return carry

        lax.fori_loop(0, n_rows // _NB - 1, step, 0)
        i = n_rows - _NB
        for b in range(_NB):
            gwait(b)
            sstart(i + b, b)
        for b in range(_NB):
            swait(b)

        plsc.subcore_barrier()

        @pl.when(cid == 0)
        def _():
            pltpu.sync_copy(acc.at[pl.ds(r0, chunk)],
                            out0_hbm.at[pl.ds(r0, chunk)])

        @pl.when(cid == 1)
        def _():
            pltpu.sync_copy(acc.at[pl.ds(r0, chunk)],
                            out1_hbm.at[pl.ds(r0, chunk)])

    return prop


@functools.lru_cache(maxsize=None)
def _make_deg(n_pad, h, n_rows):
    assert n_rows % 8 == 0
    chunk = n_pad // _NS
    assert chunk % _G == 0
    mesh = plsc.VectorSubcoreMesh(core_axis_name="c", subcore_axis_name="s")

    @functools.partial(
        pl.kernel,
        mesh=mesh,
        out_type=[
            jax.ShapeDtypeStruct((n_pad, h), jnp.float32),
            jax.ShapeDtypeStruct((n_pad, h), jnp.float32),
        ],
        scratch_types=[
            pltpu.VMEM((n_rows, _G), jnp.int32),
            pltpu.VMEM((_G, h), jnp.float32),
            pltpu.VMEM_SHARED((n_pad, h), jnp.float32),
            pltpu.SemaphoreType.DMA,
            pltpu.SemaphoreType.DMA,
        ],
        compiler_params=pltpu.CompilerParams(use_tc_tiling_on_sc=False),
    )
    def deg(edges_hbm, out0_hbm, out1_hbm, dst_a, ones_v, acc, isem, sem):
        cid = lax.axis_index("c")
        sid = lax.axis_index("s")
        wid = cid * _NS + sid
        r0 = sid * chunk
        row_base = wid * n_rows
        dst_sl = edges_hbm.at[1, pl.ds(row_base, n_rows)]
        pltpu.async_copy(dst_sl, dst_a, isem)
        one = jnp.full((h,), 1.0, jnp.float32)
        for i in range(_G):
            ones_v[i, :] = one
        for t in range(chunk // _G):
            pltpu.sync_copy(ones_v, acc.at[pl.ds(r0 + t * _G, _G)])
        pltpu.make_async_copy(dst_sl, dst_a, isem).wait()
        plsc.subcore_barrier()

        def step(j, carry):
            base = j * 8
            for t in range(8):
                pltpu.async_copy(ones_v, acc.at[dst_a.at[base + t]], sem,
                                 add=True)
            for t in range(8):
                pltpu.make_async_copy(ones_v, acc.at[dst_a.at[0]], sem).wait()
            return carry

        lax.fori_loop(0, n_rows // 8, step, 0)
        plsc.subcore_barrier()

        @pl.when(cid == 0)
        def _():
            pltpu.sync_copy(acc.at[pl.ds(r0, chunk)],
                            out0_hbm.at[pl.ds(r0, chunk)])

        @pl.when(cid == 1)
        def _():
            pltpu.sync_copy(acc.at[pl.ds(r0, chunk)],
                            out1_hbm.at[pl.ds(r0, chunk)])

    return deg


def _qrsqrt(x):
    i = lax.bitcast_convert_type(x, jnp.int32)
    i = jnp.int32(0x5F3759DF) - (i >> 1)
    y = lax.bitcast_convert_type(i, jnp.float32)
    for _ in range(3):
        y = y * (1.5 - 0.5 * x * y * y)
    return y


@functools.lru_cache(maxsize=None)
def _make_map(n_pad, h, kind):
    rpw = n_pad // (_NC * _NS)
    slabs = {"g1": 3, "g2": 5, "q": 5}[kind]
    vec_in = 1 if kind == "g2" else 0
    mesh = plsc.VectorSubcoreMesh(core_axis_name="c", subcore_axis_name="s")
    scratch = ([pltpu.VMEM((rpw, h), jnp.float32)] * slabs
               + ([pltpu.VMEM((h,), jnp.float32)] if vec_in else [])
               + [pltpu.VMEM((rpw, h), jnp.float32), pltpu.SemaphoreType.DMA])

    @functools.partial(
        pl.kernel,
        mesh=mesh,
        out_type=jax.ShapeDtypeStruct((n_pad, h), jnp.float32),
        scratch_types=scratch,
        compiler_params=pltpu.CompilerParams(use_tc_tiling_on_sc=False),
    )
    def mapk(*refs):
        n_in = slabs + vec_in
        ins_hbm = refs[:n_in]
        out_hbm = refs[n_in]
        rest = refs[n_in + 1:]
        slab_v = rest[:slabs]
        bvec_v = rest[slabs] if vec_in else None
        outb = rest[slabs + vec_in]
        sem = rest[slabs + vec_in + 1]
        cid = lax.axis_index("c")
        sid = lax.axis_index("s")
        r0 = (cid * _NS + sid) * rpw
        for k in range(slabs):
            pltpu.async_copy(ins_hbm[k].at[pl.ds(r0, rpw)], slab_v[k], sem)
        if vec_in:
            pltpu.async_copy(ins_hbm[slabs], bvec_v, sem)
        for k in range(slabs):
            pltpu.make_async_copy(ins_hbm[k].at[pl.ds(r0, rpw)], slab_v[k],
                                  sem).wait()
        if vec_in:
            pltpu.make_async_copy(ins_hbm[slabs], bvec_v, sem).wait()

        def row(i, carry):
            vals = [s[i, :] for s in slab_v]
            if kind == "g1":
                z, d0, d1 = vals
                outb[i, :] = z * _qrsqrt(d0 + d1 - 1.0)
            elif kind == "g2":
                a0, a1, g1, d0, d1 = vals
                dis = _qrsqrt(d0 + d1 - 1.0)
                h1 = jnp.maximum(dis * (a0 + a1 - g1) + bvec_v[...], 0.0)
                outb[i, :] = dis * h1
            else:
                s0, s1, g2, d0, d1 = vals
                outb[i, :] = _qrsqrt(d0 + d1 - 1.0) * (s0 + s1 - g2)
            return carry

        lax.fori_loop(0, rpw, row, 0)
        pltpu.sync_copy(outb, out_hbm.at[pl.ds(r0, rpw)])

    return mapk


def _tcz_body(x_ref, w_ref, z_ref):
    z_ref[...] = jnp.dot(x_ref[...], w_ref[...],
                         preferred_element_type=jnp.float32)


def _tc3_body(q_ref, w2_ref, b2_ref, out_ref):
    c = out_ref.shape[1]
    logits = jnp.dot(q_ref[...], w2_ref[...],
                     preferred_element_type=jnp.float32)
    logits = logits + b2_ref[...]
    m = jnp.max(logits, axis=-1, keepdims=True)
    e = jnp.exp(logits - m)
    ls = (logits - m) - jnp.log(jnp.sum(e, axis=-1, keepdims=True))
    out_ref[...] = ls[:, :c]


def _row_spec(blk, w):
    return pl.BlockSpec((blk, w), lambda i: (i, 0))


def _full_spec(a, b):
    return pl.BlockSpec((a, b), lambda i: (0, 0))


def kernel(x, edge_index, W1, b1, W2, b2):
    n, d = x.shape
    h = W1.shape[1]
    c = W2.shape[1]
    e = edge_index.shape[1]

    epw = _G * _NC * _NS
    n_rows = (e + epw - 1) // epw
    n_rows = ((n_rows + _NB - 1) // _NB) * _NB
    ep = n_rows * epw
    n_pad = ((n + _NS * 8 - 1) // (_NS * 8)) * (_NS * 8)
    if n_pad - n < _G and ep > e:
        n_pad += _NS * 8
    pad_e = ep - e

    edges = edge_index
    if pad_e:
        src_pad = (np.arange(pad_e) * 131) % n
        trash = max(n_pad - n, 1)
        dst_pad = n + (np.arange(pad_e) % trash)
        pad2 = jnp.asarray(np.stack([src_pad, dst_pad]), dtype=jnp.int32)
        edges = jnp.concatenate([edges, pad2], axis=1)
    edges = edges.reshape(2, ep // _G, _G)

    prop = _make_prop(n_pad, h, n_rows)
    degk = _make_deg(n_pad, h, n_rows)
    map_g1 = _make_map(n_pad, h, "g1")
    map_g2 = _make_map(n_pad, h, "g2")
    map_q = _make_map(n_pad, h, "q")

    f32 = jnp.float32

    d0, d1 = degk(edges)

    blk = n_pad // 8
    z1 = pl.pallas_call(
        _tcz_body,
        grid=(8,),
        in_specs=[_row_spec(blk, d), _full_spec(d, h)],
        out_specs=_row_spec(blk, h),
        out_shape=jax.ShapeDtypeStruct((n_pad, h), f32),
    )(x, W1)

    g1 = map_g1(z1, d0, d1)
    a0, a1 = prop(edges, g1)
    g2 = map_g2(a0, a1, g1, d0, d1, b1)
    s0, s1 = prop(edges, g2)
    q = map_q(s0, s1, g2, d0, d1)

    cp = 8
    w2p = jnp.zeros((h, cp), f32).at[:, :c].set(W2)
    b2p = jnp.full((1, cp), -1e30, f32).at[0, :c].set(b2)
    blk2 = n // 10
    out = pl.pallas_call(
        _tc3_body,
        grid=(10,),
        in_specs=[_row_spec(blk2, h), _full_spec(h, cp), _full_spec(1, cp)],
        out_specs=_row_spec(blk2, c),
        out_shape=jax.ShapeDtypeStruct((n, c), f32),
    )(q, w2p, b2p)

    return out


# speedup vs baseline: 1.0370x; 1.0370x over previous
import functools

import numpy as np
import jax
import jax.numpy as jnp
from jax import lax
from jax.experimental import pallas as pl
from jax.experimental.pallas import tpu as pltpu
from jax.experimental.pallas import tpu_sc as plsc

_NC = 2
_NS = 16
_G = 128
_NB = 8


@functools.lru_cache(maxsize=None)
def _make_prop(n_pad, h, n_rows):
    assert n_rows % _NB == 0
    chunk = n_pad // _NS
    mesh = plsc.VectorSubcoreMesh(core_axis_name="c", subcore_axis_name="s")

    @functools.partial(
        pl.kernel,
        mesh=mesh,
        out_type=[
            jax.ShapeDtypeStruct((n_pad, h), jnp.float32),
            jax.ShapeDtypeStruct((n_pad, h), jnp.float32),
        ],
        scratch_types=[
            pltpu.VMEM((n_rows, _G), jnp.int32),
            pltpu.VMEM((n_rows, _G), jnp.int32),
            pltpu.VMEM((_NB, _G, h), jnp.float32),
            pltpu.VMEM_SHARED((n_pad, h), jnp.float32),
            pltpu.SemaphoreType.DMA((_NB,)),
            pltpu.SemaphoreType.DMA((_NB,)),
        ],
        compiler_params=pltpu.CompilerParams(use_tc_tiling_on_sc=False),
    )
    def prop(edges_hbm, g_hbm, out0_hbm, out1_hbm,
             src_a, dst_a, rows, acc, semg, sems):
        cid = lax.axis_index("c")
        sid = lax.axis_index("s")
        wid = cid * _NS + sid
        r0 = sid * chunk
        row_base = wid * n_rows
        isem = semg.at[0]
        src_sl = edges_hbm.at[0, pl.ds(row_base, n_rows)]
        dst_sl = edges_hbm.at[1, pl.ds(row_base, n_rows)]
        pltpu.async_copy(src_sl, src_a, isem)
        pltpu.async_copy(dst_sl, dst_a, isem)
        pltpu.sync_copy(g_hbm.at[pl.ds(r0, chunk)], acc.at[pl.ds(r0, chunk)])
        pltpu.make_async_copy(src_sl, src_a, isem).wait()
        pltpu.make_async_copy(dst_sl, dst_a, isem).wait()
        plsc.subcore_barrier()

        def gstart(i, b):
            pltpu.async_copy(g_hbm.at[src_a.at[i]], rows.at[b], semg.at[b])

        def gwait(b):
            pltpu.make_async_copy(g_hbm.at[src_a.at[0]], rows.at[b],
                                  semg.at[b]).wait()

        def sstart(i, b):
            pltpu.async_copy(rows.at[b], acc.at[dst_a.at[i]], sems.at[b],
                             add=True)

        def swait(b):
            pltpu.make_async_copy(rows.at[b], acc.at[dst_a.at[0]],
                                  sems.at[b]).wait()

        for b in range(_NB):
            gstart(b, b)

        def step(k, carry):
            i = _NB * k
            for b in range(_NB):
                gwait(b)
                sstart(i + b, b)
            for b in range(_NB):
                swait(b)
                gstart(i + _NB + b, b)
            return carry

        lax.fori_loop(0, n_rows // _NB - 1, step, 0)
        i = n_rows - _NB
        for b in range(_NB):
            gwait(b)
            sstart(i + b, b)
        for b in range(_NB):
            swait(b)

        plsc.subcore_barrier()

        @pl.when(cid == 0)
        def _():
            pltpu.sync_copy(acc.at[pl.ds(r0, chunk)],
                            out0_hbm.at[pl.ds(r0, chunk)])

        @pl.when(cid == 1)
        def _():
            pltpu.sync_copy(acc.at[pl.ds(r0, chunk)],
                            out1_hbm.at[pl.ds(r0, chunk)])

    return prop


@functools.lru_cache(maxsize=None)
def _make_deg(n_pad, h, n_rows):
    assert n_rows % 8 == 0
    chunk = n_pad // _NS
    assert chunk % _G == 0
    mesh = plsc.VectorSubcoreMesh(core_axis_name="c", subcore_axis_name="s")

    @functools.partial(
        pl.kernel,
        mesh=mesh,
        out_type=[
            jax.ShapeDtypeStruct((n_pad, h), jnp.float32),
            jax.ShapeDtypeStruct((n_pad, h), jnp.float32),
        ],
        scratch_types=[
            pltpu.VMEM((n_rows, _G), jnp.int32),
            pltpu.VMEM((_G, h), jnp.float32),
            pltpu.VMEM_SHARED((n_pad, h), jnp.float32),
            pltpu.SemaphoreType.DMA,
            pltpu.SemaphoreType.DMA,
        ],
        compiler_params=pltpu.CompilerParams(use_tc_tiling_on_sc=False),
    )
    def deg(edges_hbm, out0_hbm, out1_hbm, dst_a, ones_v, acc, isem, sem):
        cid = lax.axis_index("c")
        sid = lax.axis_index("s")
        wid = cid * _NS + sid
        r0 = sid * chunk
        row_base = wid * n_rows
        dst_sl = edges_hbm.at[1, pl.ds(row_base, n_rows)]
        pltpu.async_copy(dst_sl, dst_a, isem)
        one = jnp.full((h,), 1.0, jnp.float32)
        for i in range(_G):
            ones_v[i, :] = one
        for t in range(chunk // _G):
            pltpu.sync_copy(ones_v, acc.at[pl.ds(r0 + t * _G, _G)])
        pltpu.make_async_copy(dst_sl, dst_a, isem).wait()
        plsc.subcore_barrier()

        def step(j, carry):
            base = j * 8
            for t in range(8):
                pltpu.async_copy(ones_v, acc.at[dst_a.at[base + t]], sem,
                                 add=True)
            for t in range(8):
                pltpu.make_async_copy(ones_v, acc.at[dst_a.at[0]], sem).wait()
            return carry

        lax.fori_loop(0, n_rows // 8, step, 0)
        plsc.subcore_barrier()

        @pl.when(cid == 0)
        def _():
            pltpu.sync_copy(acc.at[pl.ds(r0, chunk)],
                            out0_hbm.at[pl.ds(r0, chunk)])

        @pl.when(cid == 1)
        def _():
            pltpu.sync_copy(acc.at[pl.ds(r0, chunk)],
                            out1_hbm.at[pl.ds(r0, chunk)])

    return deg


def _qrsqrt(x):
    i = lax.bitcast_convert_type(x, jnp.int32)
    i = jnp.int32(0x5F3759DF) - (i >> 1)
    y = lax.bitcast_convert_type(i, jnp.float32)
    for _ in range(3):
        y = y * (1.5 - 0.5 * x * y * y)
    return y


@functools.lru_cache(maxsize=None)
def _make_map(n_pad, h, kind):
    rpw = n_pad // (_NC * _NS)
    slabs = {"g1": 3, "g2": 5, "q": 5}[kind]
    vec_in = 1 if kind == "g2" else 0
    packed = False
    ppr = 128 // h
    out_sds = (jax.ShapeDtypeStruct((n_pad // ppr, ppr * h), jnp.float32)
               if packed else jax.ShapeDtypeStruct((n_pad, h), jnp.float32))
    outb_t = (pltpu.VMEM((rpw // ppr, ppr * h), jnp.float32) if packed
              else pltpu.VMEM((rpw, h), jnp.float32))
    mesh = plsc.VectorSubcoreMesh(core_axis_name="c", subcore_axis_name="s")
    scratch = ([pltpu.VMEM((rpw, h), jnp.float32)] * slabs
               + ([pltpu.VMEM((h,), jnp.float32)] if vec_in else [])
               + [outb_t, pltpu.SemaphoreType.DMA])

    @functools.partial(
        pl.kernel,
        mesh=mesh,
        out_type=out_sds,
        scratch_types=scratch,
        compiler_params=pltpu.CompilerParams(use_tc_tiling_on_sc=False),
    )
    def mapk(*refs):
        n_in = slabs + vec_in
        ins_hbm = refs[:n_in]
        out_hbm = refs[n_in]
        rest = refs[n_in + 1:]
        slab_v = rest[:slabs]
        bvec_v = rest[slabs] if vec_in else None
        outb = rest[slabs + vec_in]
        sem = rest[slabs + vec_in + 1]
        cid = lax.axis_index("c")
        sid = lax.axis_index("s")
        r0 = (cid * _NS + sid) * rpw
        for k in range(slabs):
            pltpu.async_copy(ins_hbm[k].at[pl.ds(r0, rpw)], slab_v[k], sem)
        if vec_in:
            pltpu.async_copy(ins_hbm[slabs], bvec_v, sem)
        for k in range(slabs):
            pltpu.make_async_copy(ins_hbm[k].at[pl.ds(r0, rpw)], slab_v[k],
                                  sem).wait()
        if vec_in:
            pltpu.make_async_copy(ins_hbm[slabs], bvec_v, sem).wait()

        def val(i):
            vals = [s[i, :] for s in slab_v]
            if kind == "g1":
                z, d0, d1 = vals
                return z * _qrsqrt(d0 + d1 - 1.0)
            if kind == "g2":
                a0, a1, g1, d0, d1 = vals
                dis = _qrsqrt(d0 + d1 - 1.0)
                h1 = jnp.maximum(dis * (a0 + a1 - g1) + bvec_v[...], 0.0)
                return dis * h1
            s0, s1, g2, d0, d1 = vals
            return _qrsqrt(d0 + d1 - 1.0) * (s0 + s1 - g2)

        if packed:
            def row(j, carry):
                for p in range(ppr):
                    outb[j, p * h:(p + 1) * h] = val(j * ppr + p)
                return carry

            lax.fori_loop(0, rpw // ppr, row, 0)
            pltpu.sync_copy(outb, out_hbm.at[pl.ds(r0 // ppr, rpw // ppr)])
        else:
            def row(i, carry):
                outb[i, :] = val(i)
                return carry

            lax.fori_loop(0, rpw, row, 0)
            pltpu.sync_copy(outb, out_hbm.at[pl.ds(r0, rpw)])

    return mapk


def _tcz_body(x_ref, w_ref, z_ref):
    z_ref[...] = jnp.dot(x_ref[...], w_ref[...],
                         preferred_element_type=jnp.float32)


def _tc3_body(q_ref, w2_ref, b2_ref, out_ref):
    c = out_ref.shape[1]
    logits = jnp.dot(q_ref[...], w2_ref[...],
                     preferred_element_type=jnp.float32)
    logits = logits + b2_ref[...]
    m = jnp.max(logits, axis=-1, keepdims=True)
    e = jnp.exp(logits - m)
    ls = (logits - m) - jnp.log(jnp.sum(e, axis=-1, keepdims=True))
    out_ref[...] = ls[:, :c]


def _row_spec(blk, w):
    return pl.BlockSpec((blk, w), lambda i: (i, 0))


def _full_spec(a, b):
    return pl.BlockSpec((a, b), lambda i: (0, 0))


def kernel(x, edge_index, W1, b1, W2, b2):
    n, d = x.shape
    h = W1.shape[1]
    c = W2.shape[1]
    e = edge_index.shape[1]

    epw = _G * _NC * _NS
    n_rows = (e + epw - 1) // epw
    n_rows = ((n_rows + _NB - 1) // _NB) * _NB
    ep = n_rows * epw
    n_pad = ((n + _NS * 8 - 1) // (_NS * 8)) * (_NS * 8)
    if n_pad - n < _G and ep > e:
        n_pad += _NS * 8
    pad_e = ep - e

    edges = edge_index
    if pad_e:
        src_pad = (np.arange(pad_e) * 131) % n
        trash = max(n_pad - n, 1)
        dst_pad = n + (np.arange(pad_e) % trash)
        pad2 = jnp.asarray(np.stack([src_pad, dst_pad]), dtype=jnp.int32)
        edges = jnp.concatenate([edges, pad2], axis=1)
    edges = edges.reshape(2, ep // _G, _G)

    prop = _make_prop(n_pad, h, n_rows)
    degk = _make_deg(n_pad, h, n_rows)
    map_g1 = _make_map(n_pad, h, "g1")
    map_g2 = _make_map(n_pad, h, "g2")
    map_q = _make_map(n_pad, h, "q")

    f32 = jnp.float32

    d0, d1 = degk(edges)

    blk = n_pad // 8
    z1 = pl.pallas_call(
        _tcz_body,
        grid=(8,),
        in_specs=[_row_spec(blk, d), _full_spec(d, h)],
        out_specs=_row_spec(blk, h),
        out_shape=jax.ShapeDtypeStruct((n_pad, h), f32),
    )(x, W1)

    g1 = map_g1(z1, d0, d1)
    a0, a1 = prop(edges, g1)
    g2 = map_g2(a0, a1, g1, d0, d1, b1)
    s0, s1 = prop(edges, g2)
    q = map_q(s0, s1, g2, d0, d1)

    cp = 8
    w2p = jnp.zeros((h, cp), f32).at[:, :c].set(W2)
    b2p = jnp.full((1, cp), -1e30, f32).at[0, :c].set(b2)
    blk2 = n // 10
    out = pl.pallas_call(
        _tc3_body,
        grid=(10,),
        in_specs=[_row_spec(blk2, h), _full_spec(h, cp), _full_spec(1, cp)],
        out_specs=_row_spec(blk2, c),
        out_shape=jax.ShapeDtypeStruct((n, c), f32),
    )(q, w2p, b2p)

    return out
